# Initial kernel scaffold; baseline (speedup 1.0000x reference)
import jax
import jax.numpy as jnp
from jax.experimental import pallas as pl


def kernel(query, key, value, block_index):
    raise NotImplementedError("write your pallas kernel here")


# speedup vs baseline: 1.2876x; 1.2876x over previous
import functools

import jax
import jax.numpy as jnp
from jax.experimental import pallas as pl
from jax.experimental.pallas import tpu as pltpu


def _attn_body(n_blocks, k_blocks, bs, scale,
               bi_ref, q_ref, k_ref, v_ref, o_ref):
    n = pl.program_id(1)
    q = q_ref[0]
    kg = []
    vg = []
    for j in range(k_blocks):
        idx = bi_ref[n * k_blocks + j]
        kg.append(k_ref[0, pl.ds(idx * bs, bs), :])
        vg.append(v_ref[0, pl.ds(idx * bs, bs), :])
    kg = jnp.concatenate(kg, axis=0)
    vg = jnp.concatenate(vg, axis=0)
    s = jax.lax.dot_general(q, kg, (((1,), (1,)), ((), ())),
                            preferred_element_type=jnp.float32) * scale
    m = jnp.max(s, axis=1, keepdims=True)
    e = jnp.exp(s - m)
    p = e / jnp.sum(e, axis=1, keepdims=True)
    o_ref[0] = jax.lax.dot_general(p, vg, (((1,), (0,)), ((), ())),
                                   preferred_element_type=jnp.float32)


def kernel(query, key, value, block_index):
    B, H, S, Dh = query.shape
    n_blocks, k_blocks = block_index.shape
    bs = S // n_blocks
    BH = B * H
    scale = 1.0 / float(Dh) ** 0.5

    q3 = query.reshape(BH, S, Dh)
    k3 = key.reshape(BH, S, Dh)
    v3 = value.reshape(BH, S, Dh)
    bi = block_index.reshape(-1).astype(jnp.int32)

    body = functools.partial(_attn_body, n_blocks, k_blocks, bs, scale)
    out = pl.pallas_call(
        body,
        grid_spec=pltpu.PrefetchScalarGridSpec(
            num_scalar_prefetch=1,
            grid=(BH, n_blocks),
            in_specs=[
                pl.BlockSpec((1, bs, Dh), lambda bh, n, bi_ref: (bh, n, 0)),
                pl.BlockSpec((1, S, Dh), lambda bh, n, bi_ref: (bh, 0, 0)),
                pl.BlockSpec((1, S, Dh), lambda bh, n, bi_ref: (bh, 0, 0)),
            ],
            out_specs=pl.BlockSpec((1, bs, Dh), lambda bh, n, bi_ref: (bh, n, 0)),
        ),
        out_shape=jax.ShapeDtypeStruct((BH, S, Dh), jnp.float32),
    )(bi, q3, k3, v3)
    return out.reshape(B, H, S, Dh)


# speedup vs baseline: 2.2896x; 1.7782x over previous
import functools

import jax
import jax.numpy as jnp
from jax.experimental import pallas as pl
from jax.experimental.pallas import tpu as pltpu


def _attn_body(n_blocks, k_blocks, bs, scale, blocks_per_step,
               bi_ref, q_ref, k_ref, v_ref, o_ref):
    g = pl.program_id(1)
    for t in range(blocks_per_step):
        n = g * blocks_per_step + t
        q = q_ref[0, pl.ds(t * bs, bs), :]
        kg = []
        vg = []
        for j in range(k_blocks):
            idx = bi_ref[n * k_blocks + j]
            kg.append(k_ref[0, pl.ds(idx * bs, bs), :])
            vg.append(v_ref[0, pl.ds(idx * bs, bs), :])
        kg = jnp.concatenate(kg, axis=0)
        vg = jnp.concatenate(vg, axis=0)
        s = jax.lax.dot_general(q, kg, (((1,), (1,)), ((), ())),
                                preferred_element_type=jnp.float32) * scale
        m = jnp.max(s, axis=1, keepdims=True)
        e = jnp.exp(s - m)
        p = e / jnp.sum(e, axis=1, keepdims=True)
        o_ref[0, pl.ds(t * bs, bs), :] = jax.lax.dot_general(
            p, vg, (((1,), (0,)), ((), ())),
            preferred_element_type=jnp.float32)


def kernel(query, key, value, block_index):
    B, H, S, Dh = query.shape
    n_blocks, k_blocks = block_index.shape
    bs = S // n_blocks
    BH = B * H
    scale = 1.0 / float(Dh) ** 0.5

    q3 = query.reshape(BH, S, Dh)
    k3 = key.reshape(BH, S, Dh)
    v3 = value.reshape(BH, S, Dh)
    bi = block_index.reshape(-1).astype(jnp.int32)

    bps = 4
    body = functools.partial(_attn_body, n_blocks, k_blocks, bs, scale, bps)
    out = pl.pallas_call(
        body,
        grid_spec=pltpu.PrefetchScalarGridSpec(
            num_scalar_prefetch=1,
            grid=(BH, n_blocks // bps),
            in_specs=[
                pl.BlockSpec((1, bps * bs, Dh), lambda bh, g, bi_ref: (bh, g, 0)),
                pl.BlockSpec((1, S, Dh), lambda bh, g, bi_ref: (bh, 0, 0)),
                pl.BlockSpec((1, S, Dh), lambda bh, g, bi_ref: (bh, 0, 0)),
            ],
            out_specs=pl.BlockSpec((1, bps * bs, Dh), lambda bh, g, bi_ref: (bh, g, 0)),
        ),
        out_shape=jax.ShapeDtypeStruct((BH, S, Dh), jnp.float32),
    )(bi, q3, k3, v3)
    return out.reshape(B, H, S, Dh)


# speedup vs baseline: 2.4674x; 1.0777x over previous
import functools

import jax
import jax.numpy as jnp
from jax.experimental import pallas as pl
from jax.experimental.pallas import tpu as pltpu


def _attn_body(n_blocks, k_blocks, bs, scale, blocks_per_step,
               bi_ref, q_ref, k_ref, v_ref, o_ref):
    g = pl.program_id(1)
    for t in range(blocks_per_step):
        n = g * blocks_per_step + t
        q = q_ref[0, pl.ds(t * bs, bs), :]
        kg = []
        vg = []
        for j in range(k_blocks):
            idx = bi_ref[n * k_blocks + j]
            kg.append(k_ref[0, pl.ds(idx * bs, bs), :])
            vg.append(v_ref[0, pl.ds(idx * bs, bs), :])
        kg = jnp.concatenate(kg, axis=0)
        vg = jnp.concatenate(vg, axis=0)
        s = jax.lax.dot_general(q, kg, (((1,), (1,)), ((), ())),
                                preferred_element_type=jnp.float32) * scale
        m = jnp.max(s, axis=1, keepdims=True)
        e = jnp.exp(s - m)
        p = e / jnp.sum(e, axis=1, keepdims=True)
        o_ref[0, pl.ds(t * bs, bs), :] = jax.lax.dot_general(
            p, vg, (((1,), (0,)), ((), ())),
            preferred_element_type=jnp.float32)


def kernel(query, key, value, block_index):
    B, H, S, Dh = query.shape
    n_blocks, k_blocks = block_index.shape
    bs = S // n_blocks
    BH = B * H
    scale = 1.0 / float(Dh) ** 0.5

    q3 = query.reshape(BH, S, Dh)
    k3 = key.reshape(BH, S, Dh)
    v3 = value.reshape(BH, S, Dh)
    bi = block_index.reshape(-1).astype(jnp.int32)

    bps = 8
    body = functools.partial(_attn_body, n_blocks, k_blocks, bs, scale, bps)
    out = pl.pallas_call(
        body,
        grid_spec=pltpu.PrefetchScalarGridSpec(
            num_scalar_prefetch=1,
            grid=(BH, n_blocks // bps),
            in_specs=[
                pl.BlockSpec((1, bps * bs, Dh), lambda bh, g, bi_ref: (bh, g, 0)),
                pl.BlockSpec((1, S, Dh), lambda bh, g, bi_ref: (bh, 0, 0)),
                pl.BlockSpec((1, S, Dh), lambda bh, g, bi_ref: (bh, 0, 0)),
            ],
            out_specs=pl.BlockSpec((1, bps * bs, Dh), lambda bh, g, bi_ref: (bh, g, 0)),
        ),
        out_shape=jax.ShapeDtypeStruct((BH, S, Dh), jnp.float32),
    )(bi, q3, k3, v3)
    return out.reshape(B, H, S, Dh)
